# baseline (speedup 1.0000x reference)
import functools
import math

import jax
import jax.numpy as jnp
from jax import lax
from jax.experimental import pallas as pl
from jax.experimental.pallas import tpu as pltpu
from jax.experimental.pallas import tpu_sc as plsc

S = 2048
D = 768
F = 2048
E = 7
TT = 256
TF = 512
NF = F // TF
NT = S // TT
GMAX = 14
PMAX = GMAX * TT
NW = 32
CHUNK = PMAX // NW
NEG = -1e30


def _gelu_exact(t):
    return t * 0.5 * (1.0 + jax.lax.erf(t * (1.0 / math.sqrt(2.0))))


def _gs_body(x_ref, wg_ref, bg_ref, w1_ref, b1_ref, w2_ref, b2_ref,
             base_ref, eid_ref, w_ref, xbf_ref, acc_ref):
    f = pl.program_id(0)
    t = pl.program_id(1)

    @pl.when((f == 0) & (t == 0))
    def _gating():
        logits = jnp.dot(x_ref[...], wg_ref[...],
                         preferred_element_type=jnp.float32) + bg_ref[...]
        lane = lax.broadcasted_iota(jnp.int32, (S, 128), 1)
        valid = lane < E
        lm = jnp.where(valid, logits, NEG)
        m = jnp.max(lm, axis=1, keepdims=True)
        p = jnp.where(valid, jnp.exp(lm - m), 0.0)
        w = 1.0 / jnp.sum(p, axis=1, keepdims=True)
        mn = jnp.min(jnp.where(lm == m, lane, 127), axis=1, keepdims=True)
        eid_ref[...] = mn
        w_ref[...] = w

    rows = pl.ds(t * TT, TT)
    xt = x_ref[rows, :]
    xt_b = xt.astype(jnp.bfloat16)

    @pl.when(f == 0)
    def _xbf():
        xbf_ref[rows, :] = xt_b

    h = jnp.dot(xt_b, w1_ref[...].astype(jnp.bfloat16),
                preferred_element_type=jnp.float32) + b1_ref[...]
    h = _gelu_exact(h).astype(jnp.bfloat16)
    y = jnp.dot(h, w2_ref[...].astype(jnp.bfloat16),
                preferred_element_type=jnp.float32)
    y = jnp.where(f == 0, y + b2_ref[...], y)

    @pl.when(f == 0)
    def _init():
        acc_ref[rows, :] = y

    @pl.when(f != 0)
    def _acc():
        acc_ref[rows, :] += y

    @pl.when(f == NF - 1)
    def _fin():
        base_ref[rows, :] = acc_ref[rows, :] + xt


def _gate_shared(x2, wg_p, bg_p, W1_sh, b1_sh, W2_sh, b2_sh):
    return pl.pallas_call(
        _gs_body,
        grid=(NF, NT),
        in_specs=[pl.BlockSpec((S, D), lambda f, t: (0, 0)),
                  pl.BlockSpec((D, 128), lambda f, t: (0, 0)),
                  pl.BlockSpec((1, 128), lambda f, t: (0, 0)),
                  pl.BlockSpec((D, TF), lambda f, t: (0, f)),
                  pl.BlockSpec((1, TF), lambda f, t: (0, f)),
                  pl.BlockSpec((TF, D), lambda f, t: (f, 0)),
                  pl.BlockSpec((1, D), lambda f, t: (0, 0))],
        out_specs=[pl.BlockSpec((S, D), lambda f, t: (0, 0)),
                   pl.BlockSpec((S, 1), lambda f, t: (0, 0)),
                   pl.BlockSpec((S, 1), lambda f, t: (0, 0)),
                   pl.BlockSpec((S, D), lambda f, t: (0, 0))],
        out_shape=[jax.ShapeDtypeStruct((S, D), jnp.float32),
                   jax.ShapeDtypeStruct((S, 1), jnp.int32),
                   jax.ShapeDtypeStruct((S, 1), jnp.float32),
                   jax.ShapeDtypeStruct((S, D), jnp.bfloat16)],
        scratch_shapes=[pltpu.VMEM((S, D), jnp.float32)],
        compiler_params=pltpu.CompilerParams(
            dimension_semantics=("arbitrary", "arbitrary")),
    )(x2, wg_p, bg_p, W1_sh, b1_sh, W2_sh, b2_sh)


def _route_body(eid_hbm, wgt_hbm,
                scat_hbm, ws_hbm, te_hbm,
                eid_v, wgt_v, scat_v, ws_v, te_v):
    sid = lax.axis_index("s")
    cid = lax.axis_index("c")

    @pl.when((sid == 0) & (cid == 0))
    def _meta():
        pltpu.sync_copy(eid_hbm, eid_v)
        pltpu.sync_copy(wgt_hbm, wgt_v)

        def cnt_body(i, cs):
            ev = eid_v[pl.ds(i * 16, 16)]
            return tuple(c + jnp.sum(jnp.where(ev == e, 1, 0))
                         for e, c in enumerate(cs))
        z = jnp.int32(0)
        counts = lax.fori_loop(0, S // 16, cnt_body, (z,) * E)

        bases = [z]
        for e in range(1, E):
            pc = ((counts[e - 1] + (TT - 1)) >> 8) << 8
            bases.append(bases[e - 1] + pc)

        g_iota = lax.iota(jnp.int32, 16)
        te = jnp.zeros((16,), jnp.int32)
        for e in range(1, E):
            te = te + jnp.where(g_iota >= (bases[e] >> 8), 1, 0)
        te_v[...] = te

        def init_body(j, _):
            sl = pl.ds(j * 16, 16)
            scat_v[sl] = jnp.full((16,), S, jnp.int32)
            ws_v[sl] = jnp.zeros((16,), jnp.float32)
            return 0
        lax.fori_loop(0, PMAX // 16, init_body, 0)

        def pos_body(i, wps):
            sl = pl.ds(i * 16, 16)
            ev = eid_v[sl]
            wv = wgt_v[sl]
            tok = lax.iota(jnp.int32, 16) + i * 16
            pos = jnp.zeros((16,), jnp.int32)
            new_wps = []
            for e in range(E):
                mk = ev == e
                mi = jnp.where(mk, 1, 0)
                cs = plsc.cumsum(mi)
                pos = pos + jnp.where(mk, wps[e] + cs - mi, 0)
                new_wps.append(wps[e] + jnp.sum(mi))
            plsc.store_scatter(scat_v, [pos], tok)
            plsc.store_scatter(ws_v, [pos], wv)
            return tuple(new_wps)
        lax.fori_loop(0, S // 16, pos_body, tuple(bases))

        pltpu.sync_copy(scat_v, scat_hbm)
        pltpu.sync_copy(ws_v, ws_hbm)
        pltpu.sync_copy(te_v, te_hbm)


def _route(eid, wgt):
    mesh = plsc.VectorSubcoreMesh(core_axis_name="c", subcore_axis_name="s")
    f = pl.kernel(
        _route_body,
        out_type=[jax.ShapeDtypeStruct((PMAX,), jnp.int32),
                  jax.ShapeDtypeStruct((PMAX,), jnp.float32),
                  jax.ShapeDtypeStruct((16,), jnp.int32)],
        mesh=mesh,
        scratch_types=[pltpu.VMEM((S,), jnp.int32),
                       pltpu.VMEM((S,), jnp.float32),
                       pltpu.VMEM((PMAX,), jnp.int32),
                       pltpu.VMEM((PMAX,), jnp.float32),
                       pltpu.VMEM((16,), jnp.int32)],
        compiler_params=pltpu.CompilerParams(needs_layout_passes=False),
    )
    return f(eid, wgt)


def _gu_body(te_ref, scat_c_ref, scat_r_ref, xbf_ref, ws_ref, base_ref,
             w1r_ref, b1r_ref, w2r_ref, b2r_ref,
             out_ref, acc_ref, xg_ref, ys_ref):
    f = pl.program_id(0)
    g = pl.program_id(1)
    rows = pl.ds(g * TT, TT)

    @pl.when(f == 0)
    def _gather():
        lanes = lax.broadcasted_iota(jnp.int32, (TT, S), 1)
        oh = (scat_c_ref[rows, :] == lanes).astype(jnp.bfloat16)
        xg_ref[rows, :] = jnp.dot(oh, xbf_ref[...],
                                  preferred_element_type=jnp.float32
                                  ).astype(jnp.bfloat16)

    @pl.when(f < NF)
    def _ffn():
        xt = xg_ref[rows, :]
        h = jnp.dot(xt, w1r_ref[0].astype(jnp.bfloat16),
                    preferred_element_type=jnp.float32) + b1r_ref[0]
        h = _gelu_exact(h).astype(jnp.bfloat16)
        y = jnp.dot(h, w2r_ref[0].astype(jnp.bfloat16),
                    preferred_element_type=jnp.float32)
        y = jnp.where(f == 0, y + b2r_ref[0], y)

        @pl.when(f == 0)
        def _init():
            acc_ref[rows, :] = y

        @pl.when(f != 0)
        def _acc():
            acc_ref[rows, :] += y

        @pl.when(f == NF - 1)
        def _fin():
            ys_ref[rows, :] = (ws_ref[rows, :]
                               * acc_ref[rows, :]).astype(jnp.bfloat16)

    @pl.when((f == NF) & (g < NT))
    def _unsort():
        tio = lax.broadcasted_iota(jnp.int32, (TT, 1), 0) + g * TT
        trows = pl.ds(g * TT, TT)
        acc = base_ref[trows, :]
        for sb in range(GMAX):
            oh = (scat_r_ref[:, pl.ds(sb * TT, TT)] == tio).astype(jnp.bfloat16)
            acc += jnp.dot(oh, ys_ref[pl.ds(sb * TT, TT), :],
                           preferred_element_type=jnp.float32)
        out_ref[trows, :] = acc


def _grouped_unsort(te, scat_col, scat_row, xbf, ws2, base,
                    W1_r, b1_r, W2_r, b2_r):
    def fw(f):
        return jnp.minimum(f, NF - 1)

    grid_spec = pltpu.PrefetchScalarGridSpec(
        num_scalar_prefetch=1,
        grid=(NF + 1, GMAX),
        in_specs=[pl.BlockSpec((PMAX, 1), lambda f, g, te: (0, 0)),
                  pl.BlockSpec((1, PMAX), lambda f, g, te: (0, 0)),
                  pl.BlockSpec((S, D), lambda f, g, te: (0, 0)),
                  pl.BlockSpec((PMAX, 1), lambda f, g, te: (0, 0)),
                  pl.BlockSpec((S, D), lambda f, g, te: (0, 0)),
                  pl.BlockSpec((1, D, TF), lambda f, g, te: (te[g], 0, fw(f))),
                  pl.BlockSpec((1, 1, TF), lambda f, g, te: (te[g], 0, fw(f))),
                  pl.BlockSpec((1, TF, D), lambda f, g, te: (te[g], fw(f), 0)),
                  pl.BlockSpec((1, 1, D), lambda f, g, te: (te[g], 0, 0))],
        out_specs=pl.BlockSpec((S, D), lambda f, g, te: (0, 0)),
        scratch_shapes=[pltpu.VMEM((PMAX, D), jnp.float32),
                        pltpu.VMEM((PMAX, D), jnp.bfloat16),
                        pltpu.VMEM((PMAX, D), jnp.bfloat16)],
    )
    return pl.pallas_call(
        _gu_body,
        grid_spec=grid_spec,
        out_shape=jax.ShapeDtypeStruct((S, D), jnp.float32),
        compiler_params=pltpu.CompilerParams(
            dimension_semantics=("arbitrary", "arbitrary")),
    )(te, scat_col, scat_row, xbf, ws2, base,
      W1_r, b1_r.reshape(E, 1, F), W2_r, b2_r.reshape(E, 1, D))


def kernel(x, Wg, bg, W1_sh, b1_sh, W2_sh, b2_sh, W1_r, b1_r, W2_r, b2_r):
    B = x.shape[0]
    x2 = x.reshape(S, D)
    wg_p = jnp.zeros((D, 128), jnp.float32).at[:, :E].set(Wg)
    bg_p = jnp.zeros((1, 128), jnp.float32).at[0, :E].set(bg)

    base, eid2, wgt2, xbf = _gate_shared(x2, wg_p, bg_p, W1_sh,
                                         b1_sh.reshape(1, F), W2_sh,
                                         b2_sh.reshape(1, D))
    scat, ws, te = _route(eid2.reshape(S), wgt2.reshape(S))
    out = _grouped_unsort(te, scat.reshape(PMAX, 1), scat.reshape(1, PMAX),
                          xbf, ws.reshape(PMAX, 1), base,
                          W1_r, b1_r, W2_r, b2_r)
    return out.reshape(B, S, D)


# speedup vs baseline: 1.0221x; 1.0221x over previous
import functools
import math

import jax
import jax.numpy as jnp
from jax import lax
from jax.experimental import pallas as pl
from jax.experimental.pallas import tpu as pltpu
from jax.experimental.pallas import tpu_sc as plsc

S = 2048
D = 768
F = 2048
E = 7
TT = 256
TF = 512
NF = F // TF
NT = S // TT
GMAX = 14
PMAX = GMAX * TT
NW = 32
CHUNK = PMAX // NW
NEG = -1e30


def _gelu_exact(t):
    return t * 0.5 * (1.0 + jax.lax.erf(t * (1.0 / math.sqrt(2.0))))


def _gs_body(x_ref, wg_ref, bg_ref, w1_ref, b1_ref, w2_ref, b2_ref,
             base_ref, eid_ref, w_ref, xbf_ref, acc_ref):
    f = pl.program_id(0)
    t = pl.program_id(1)

    @pl.when((f == 0) & (t == 0))
    def _gating():
        logits = jnp.dot(x_ref[...], wg_ref[...],
                         preferred_element_type=jnp.float32) + bg_ref[...]
        lane = lax.broadcasted_iota(jnp.int32, (S, 128), 1)
        valid = lane < E
        lm = jnp.where(valid, logits, NEG)
        m = jnp.max(lm, axis=1, keepdims=True)
        p = jnp.where(valid, jnp.exp(lm - m), 0.0)
        w = 1.0 / jnp.sum(p, axis=1, keepdims=True)
        mn = jnp.min(jnp.where(lm == m, lane, 127), axis=1, keepdims=True)
        eid_ref[...] = mn
        w_ref[...] = w

    rows = pl.ds(t * TT, TT)
    xt = x_ref[rows, :]
    xt_b = xt.astype(jnp.bfloat16)

    @pl.when(f == 0)
    def _xbf():
        xbf_ref[rows, :] = xt_b

    h = jnp.dot(xt_b, w1_ref[...].astype(jnp.bfloat16),
                preferred_element_type=jnp.float32) + b1_ref[...]
    h = _gelu_exact(h).astype(jnp.bfloat16)
    y = jnp.dot(h, w2_ref[...].astype(jnp.bfloat16),
                preferred_element_type=jnp.float32)
    y = jnp.where(f == 0, y + b2_ref[...], y)

    @pl.when(f == 0)
    def _init():
        acc_ref[rows, :] = y

    @pl.when(f != 0)
    def _acc():
        acc_ref[rows, :] += y

    @pl.when(f == NF - 1)
    def _fin():
        base_ref[rows, :] = acc_ref[rows, :] + xt


def _gate_shared(x2, wg_p, bg_p, W1_sh, b1_sh, W2_sh, b2_sh):
    return pl.pallas_call(
        _gs_body,
        grid=(NF, NT),
        in_specs=[pl.BlockSpec((S, D), lambda f, t: (0, 0)),
                  pl.BlockSpec((D, 128), lambda f, t: (0, 0)),
                  pl.BlockSpec((1, 128), lambda f, t: (0, 0)),
                  pl.BlockSpec((D, TF), lambda f, t: (0, f)),
                  pl.BlockSpec((1, TF), lambda f, t: (0, f)),
                  pl.BlockSpec((TF, D), lambda f, t: (f, 0)),
                  pl.BlockSpec((1, D), lambda f, t: (0, 0))],
        out_specs=[pl.BlockSpec((S, D), lambda f, t: (0, 0)),
                   pl.BlockSpec((S, 1), lambda f, t: (0, 0)),
                   pl.BlockSpec((S, 1), lambda f, t: (0, 0)),
                   pl.BlockSpec((S, D), lambda f, t: (0, 0))],
        out_shape=[jax.ShapeDtypeStruct((S, D), jnp.float32),
                   jax.ShapeDtypeStruct((S, 1), jnp.int32),
                   jax.ShapeDtypeStruct((S, 1), jnp.float32),
                   jax.ShapeDtypeStruct((S, D), jnp.bfloat16)],
        scratch_shapes=[pltpu.VMEM((S, D), jnp.float32)],
        compiler_params=pltpu.CompilerParams(
            dimension_semantics=("arbitrary", "arbitrary")),
    )(x2, wg_p, bg_p, W1_sh, b1_sh, W2_sh, b2_sh)


def _route_body(eid_hbm, wgt_hbm,
                scat_hbm, ws_hbm, te_hbm,
                eid_v, wgt_v, scat_v, ws_v, te_v):
    sid = lax.axis_index("s")
    cid = lax.axis_index("c")

    @pl.when((sid == 0) & (cid == 0))
    def _meta():
        pltpu.sync_copy(eid_hbm, eid_v)
        pltpu.sync_copy(wgt_hbm, wgt_v)

        def cnt_body(i, cs):
            ev = eid_v[pl.ds(i * 16, 16)]
            return tuple(c + jnp.sum(jnp.where(ev == e, 1, 0))
                         for e, c in enumerate(cs))
        z = jnp.int32(0)
        counts = lax.fori_loop(0, S // 16, cnt_body, (z,) * E)

        bases = [z]
        for e in range(1, E):
            pc = ((counts[e - 1] + (TT - 1)) >> 8) << 8
            bases.append(bases[e - 1] + pc)

        g_iota = lax.iota(jnp.int32, 16)
        te = jnp.zeros((16,), jnp.int32)
        for e in range(1, E):
            te = te + jnp.where(g_iota >= (bases[e] >> 8), 1, 0)
        te_v[...] = te

        def init_body(j, _):
            sl = pl.ds(j * 16, 16)
            scat_v[sl] = jnp.full((16,), S, jnp.int32)
            ws_v[sl] = jnp.zeros((16,), jnp.float32)
            return 0
        lax.fori_loop(0, PMAX // 16, init_body, 0)

        def pos_body(i, wps):
            sl = pl.ds(i * 16, 16)
            ev = eid_v[sl]
            wv = wgt_v[sl]
            tok = lax.iota(jnp.int32, 16) + i * 16
            pos = jnp.zeros((16,), jnp.int32)
            new_wps = []
            for e in range(E):
                mk = ev == e
                mi = jnp.where(mk, 1, 0)
                cs = plsc.cumsum(mi)
                pos = pos + jnp.where(mk, wps[e] + cs - mi, 0)
                new_wps.append(wps[e] + jnp.sum(mi))
            plsc.store_scatter(scat_v, [pos], tok)
            plsc.store_scatter(ws_v, [pos], wv)
            return tuple(new_wps)
        lax.fori_loop(0, S // 16, pos_body, tuple(bases))

        pltpu.sync_copy(scat_v, scat_hbm)
        pltpu.sync_copy(ws_v, ws_hbm)
        pltpu.sync_copy(te_v, te_hbm)


def _route(eid, wgt):
    mesh = plsc.VectorSubcoreMesh(core_axis_name="c", subcore_axis_name="s")
    f = pl.kernel(
        _route_body,
        out_type=[jax.ShapeDtypeStruct((PMAX,), jnp.int32),
                  jax.ShapeDtypeStruct((PMAX,), jnp.float32),
                  jax.ShapeDtypeStruct((16,), jnp.int32)],
        mesh=mesh,
        scratch_types=[pltpu.VMEM((S,), jnp.int32),
                       pltpu.VMEM((S,), jnp.float32),
                       pltpu.VMEM((PMAX,), jnp.int32),
                       pltpu.VMEM((PMAX,), jnp.float32),
                       pltpu.VMEM((16,), jnp.int32)],
        compiler_params=pltpu.CompilerParams(needs_layout_passes=False),
    )
    return f(eid, wgt)


def _group_body(te_ref, scat_ref, xbf_ref, ws_ref, w1r_ref, b1r_ref, w2r_ref,
                b2r_ref, ys_ref, acc_ref, xg_ref):
    f = pl.program_id(0)
    g = pl.program_id(1)
    rows = pl.ds(g * TT, TT)

    @pl.when(f == 0)
    def _gather():
        lanes = lax.broadcasted_iota(jnp.int32, (TT, S), 1)
        oh = (scat_ref[rows, :] == lanes).astype(jnp.bfloat16)
        xg_ref[rows, :] = jnp.dot(oh, xbf_ref[...],
                                  preferred_element_type=jnp.float32
                                  ).astype(jnp.bfloat16)

    xt = xg_ref[rows, :]
    h = jnp.dot(xt, w1r_ref[0].astype(jnp.bfloat16),
                preferred_element_type=jnp.float32) + b1r_ref[0]
    h = _gelu_exact(h).astype(jnp.bfloat16)
    y = jnp.dot(h, w2r_ref[0].astype(jnp.bfloat16),
                preferred_element_type=jnp.float32)
    y = jnp.where(f == 0, y + b2r_ref[0], y)

    @pl.when(f == 0)
    def _init():
        acc_ref[rows, :] = y

    @pl.when(f != 0)
    def _acc():
        acc_ref[rows, :] += y

    @pl.when(f == NF - 1)
    def _fin():
        ys_ref[rows, :] = (ws_ref[rows, :]
                           * acc_ref[rows, :]).astype(jnp.bfloat16)


def _grouped(te, scat_col, xbf, ws2, W1_r, b1_r, W2_r, b2_r):
    grid_spec = pltpu.PrefetchScalarGridSpec(
        num_scalar_prefetch=1,
        grid=(NF, GMAX),
        in_specs=[pl.BlockSpec((PMAX, 1), lambda f, g, te: (0, 0)),
                  pl.BlockSpec((S, D), lambda f, g, te: (0, 0)),
                  pl.BlockSpec((PMAX, 1), lambda f, g, te: (0, 0)),
                  pl.BlockSpec((1, D, TF), lambda f, g, te: (te[g], 0, f)),
                  pl.BlockSpec((1, 1, TF), lambda f, g, te: (te[g], 0, f)),
                  pl.BlockSpec((1, TF, D), lambda f, g, te: (te[g], f, 0)),
                  pl.BlockSpec((1, 1, D), lambda f, g, te: (te[g], 0, 0))],
        out_specs=pl.BlockSpec((PMAX, D), lambda f, g, te: (0, 0)),
        scratch_shapes=[pltpu.VMEM((PMAX, D), jnp.float32),
                        pltpu.VMEM((PMAX, D), jnp.bfloat16)],
    )
    return pl.pallas_call(
        _group_body,
        grid_spec=grid_spec,
        out_shape=jax.ShapeDtypeStruct((PMAX, D), jnp.bfloat16),
        compiler_params=pltpu.CompilerParams(
            dimension_semantics=("arbitrary", "arbitrary")),
    )(te, scat_col, xbf, ws2, W1_r, b1_r.reshape(E, 1, F), W2_r,
      b2_r.reshape(E, 1, D))


def _unsort_body(scat_ref, ys_ref, base_ref, out_ref):
    t = pl.program_id(0)
    tio = lax.broadcasted_iota(jnp.int32, (TT, 1), 0) + t * TT
    acc = base_ref[...]
    for s in range(GMAX):
        oh = (scat_ref[:, pl.ds(s * TT, TT)] == tio).astype(jnp.bfloat16)
        acc += jnp.dot(oh, ys_ref[pl.ds(s * TT, TT), :],
                       preferred_element_type=jnp.float32)
    out_ref[...] = acc


def _combine(scat_row, ys, base):
    return pl.pallas_call(
        _unsort_body,
        grid=(NT,),
        in_specs=[pl.BlockSpec((1, PMAX), lambda t: (0, 0)),
                  pl.BlockSpec((PMAX, D), lambda t: (0, 0)),
                  pl.BlockSpec((TT, D), lambda t: (t, 0))],
        out_specs=pl.BlockSpec((TT, D), lambda t: (t, 0)),
        out_shape=jax.ShapeDtypeStruct((S, D), jnp.float32),
        compiler_params=pltpu.CompilerParams(
            dimension_semantics=("arbitrary",)),
    )(scat_row, ys, base)


def kernel(x, Wg, bg, W1_sh, b1_sh, W2_sh, b2_sh, W1_r, b1_r, W2_r, b2_r):
    B = x.shape[0]
    x2 = x.reshape(S, D)
    wg_p = jnp.zeros((D, 128), jnp.float32).at[:, :E].set(Wg)
    bg_p = jnp.zeros((1, 128), jnp.float32).at[0, :E].set(bg)

    base, eid2, wgt2, xbf = _gate_shared(x2, wg_p, bg_p, W1_sh,
                                         b1_sh.reshape(1, F), W2_sh,
                                         b2_sh.reshape(1, D))
    scat, ws, te = _route(eid2.reshape(S), wgt2.reshape(S))
    ys = _grouped(te, scat.reshape(PMAX, 1), xbf, ws.reshape(PMAX, 1),
                  W1_r, b1_r, W2_r, b2_r)
    out = _combine(scat.reshape(1, PMAX), ys, base)
    return out.reshape(B, S, D)


# speedup vs baseline: 1.0506x; 1.0279x over previous
import functools
import math

import jax
import jax.numpy as jnp
from jax import lax
from jax.experimental import pallas as pl
from jax.experimental.pallas import tpu as pltpu
from jax.experimental.pallas import tpu_sc as plsc

S = 2048
D = 768
F = 2048
E = 7
TT = 256
TF = 512
NF = F // TF
NT = S // TT
GMAX = 14
PMAX = GMAX * TT
NW = 32
CHUNK = PMAX // NW
NEG = -1e30


def _gelu_exact(t):
    return t * 0.5 * (1.0 + jax.lax.erf(t * (1.0 / math.sqrt(2.0))))


def _gate_body(x_ref, wg_ref, bg_ref, eid_ref, w_ref):
    logits = jnp.dot(x_ref[...], wg_ref[...],
                     preferred_element_type=jnp.float32) + bg_ref[...]
    lane = lax.broadcasted_iota(jnp.int32, (S, 128), 1)
    valid = lane < E
    lm = jnp.where(valid, logits, NEG)
    m = jnp.max(lm, axis=1, keepdims=True)
    p = jnp.where(valid, jnp.exp(lm - m), 0.0)
    w = 1.0 / jnp.sum(p, axis=1, keepdims=True)
    mn = jnp.min(jnp.where(lm == m, lane, 127), axis=1, keepdims=True)
    eid_ref[...] = mn
    w_ref[...] = w


def _gate(x2, wg_p, bg_p):
    return pl.pallas_call(
        _gate_body,
        in_specs=[pl.BlockSpec((S, D), lambda: (0, 0)),
                  pl.BlockSpec((D, 128), lambda: (0, 0)),
                  pl.BlockSpec((1, 128), lambda: (0, 0))],
        out_specs=[pl.BlockSpec((S, 1), lambda: (0, 0)),
                   pl.BlockSpec((S, 1), lambda: (0, 0))],
        out_shape=[jax.ShapeDtypeStruct((S, 1), jnp.int32),
                   jax.ShapeDtypeStruct((S, 1), jnp.float32)],
    )(x2, wg_p, bg_p)


def _route_body(eid_hbm, wgt_hbm,
                scat_hbm, ws_hbm, te_hbm,
                eid_v, wgt_v, scat_v, ws_v, te_v):
    sid = lax.axis_index("s")
    cid = lax.axis_index("c")

    @pl.when((sid == 0) & (cid == 0))
    def _meta():
        pltpu.sync_copy(eid_hbm, eid_v)
        pltpu.sync_copy(wgt_hbm, wgt_v)

        def cnt_body(i, cs):
            ev = eid_v[pl.ds(i * 16, 16)]
            return tuple(c + jnp.sum(jnp.where(ev == e, 1, 0))
                         for e, c in enumerate(cs))
        z = jnp.int32(0)
        counts = lax.fori_loop(0, S // 16, cnt_body, (z,) * E)

        bases = [z]
        for e in range(1, E):
            pc = ((counts[e - 1] + (TT - 1)) >> 8) << 8
            bases.append(bases[e - 1] + pc)

        g_iota = lax.iota(jnp.int32, 16)
        te = jnp.zeros((16,), jnp.int32)
        for e in range(1, E):
            te = te + jnp.where(g_iota >= (bases[e] >> 8), 1, 0)
        te_v[...] = te

        def init_body(j, _):
            sl = pl.ds(j * 16, 16)
            scat_v[sl] = jnp.full((16,), S, jnp.int32)
            ws_v[sl] = jnp.zeros((16,), jnp.float32)
            return 0
        lax.fori_loop(0, PMAX // 16, init_body, 0)

        def pos_body(i, wps):
            sl = pl.ds(i * 16, 16)
            ev = eid_v[sl]
            wv = wgt_v[sl]
            tok = lax.iota(jnp.int32, 16) + i * 16
            pos = jnp.zeros((16,), jnp.int32)
            new_wps = []
            for e in range(E):
                mk = ev == e
                mi = jnp.where(mk, 1, 0)
                cs = plsc.cumsum(mi)
                pos = pos + jnp.where(mk, wps[e] + cs - mi, 0)
                new_wps.append(wps[e] + jnp.sum(mi))
            plsc.store_scatter(scat_v, [pos], tok)
            plsc.store_scatter(ws_v, [pos], wv)
            return tuple(new_wps)
        lax.fori_loop(0, S // 16, pos_body, tuple(bases))

        pltpu.sync_copy(scat_v, scat_hbm)
        pltpu.sync_copy(ws_v, ws_hbm)
        pltpu.sync_copy(te_v, te_hbm)


def _route(eid, wgt):
    mesh = plsc.VectorSubcoreMesh(core_axis_name="c", subcore_axis_name="s")
    f = pl.kernel(
        _route_body,
        out_type=[jax.ShapeDtypeStruct((PMAX,), jnp.int32),
                  jax.ShapeDtypeStruct((PMAX,), jnp.float32),
                  jax.ShapeDtypeStruct((16,), jnp.int32)],
        mesh=mesh,
        scratch_types=[pltpu.VMEM((S,), jnp.int32),
                       pltpu.VMEM((S,), jnp.float32),
                       pltpu.VMEM((PMAX,), jnp.int32),
                       pltpu.VMEM((PMAX,), jnp.float32),
                       pltpu.VMEM((16,), jnp.int32)],
        compiler_params=pltpu.CompilerParams(needs_layout_passes=False),
    )
    return f(eid, wgt)


def _shared_body(x_ref, w1_ref, b1_ref, w2_ref, b2_ref, base_ref, acc_ref):
    f = pl.program_id(0)
    t = pl.program_id(1)
    rows = pl.ds(t * TT, TT)
    xt = x_ref[rows, :]
    h = jnp.dot(xt.astype(jnp.bfloat16), w1_ref[...].astype(jnp.bfloat16),
                preferred_element_type=jnp.float32) + b1_ref[...]
    h = _gelu_exact(h).astype(jnp.bfloat16)
    y = jnp.dot(h, w2_ref[...].astype(jnp.bfloat16),
                preferred_element_type=jnp.float32)
    y = jnp.where(f == 0, y + b2_ref[...], y)

    @pl.when(f == 0)
    def _init():
        acc_ref[rows, :] = y

    @pl.when(f != 0)
    def _acc():
        acc_ref[rows, :] += y

    @pl.when(f == NF - 1)
    def _fin():
        base_ref[rows, :] = acc_ref[rows, :] + xt


def _shared(x2, W1_sh, b1_sh, W2_sh, b2_sh):
    return pl.pallas_call(
        _shared_body,
        grid=(NF, NT),
        in_specs=[pl.BlockSpec((S, D), lambda f, t: (0, 0)),
                  pl.BlockSpec((D, TF), lambda f, t: (0, f)),
                  pl.BlockSpec((1, TF), lambda f, t: (0, f)),
                  pl.BlockSpec((TF, D), lambda f, t: (f, 0)),
                  pl.BlockSpec((1, D), lambda f, t: (0, 0))],
        out_specs=pl.BlockSpec((S, D), lambda f, t: (0, 0)),
        out_shape=jax.ShapeDtypeStruct((S, D), jnp.float32),
        scratch_shapes=[pltpu.VMEM((S, D), jnp.float32)],
        compiler_params=pltpu.CompilerParams(
            dimension_semantics=("arbitrary", "arbitrary")),
    )(x2, W1_sh, b1_sh, W2_sh, b2_sh)


def _group_body(te_ref, scat_ref, x_ref, ws_ref, w1r_ref, b1r_ref, w2r_ref,
                b2r_ref, ys_ref, acc_ref, xb_ref, xg_ref):
    f = pl.program_id(0)
    g = pl.program_id(1)
    rows = pl.ds(g * TT, TT)

    @pl.when((f == 0) & (g == 0))
    def _cast_x():
        xb_ref[...] = x_ref[...].astype(jnp.bfloat16)

    @pl.when(f == 0)
    def _gather():
        lanes = lax.broadcasted_iota(jnp.int32, (TT, S), 1)
        oh = (scat_ref[rows, :] == lanes).astype(jnp.bfloat16)
        xg_ref[rows, :] = jnp.dot(oh, xb_ref[...],
                                  preferred_element_type=jnp.float32
                                  ).astype(jnp.bfloat16)

    xt = xg_ref[rows, :]
    h = jnp.dot(xt, w1r_ref[0].astype(jnp.bfloat16),
                preferred_element_type=jnp.float32) + b1r_ref[0]
    h = _gelu_exact(h).astype(jnp.bfloat16)
    y = jnp.dot(h, w2r_ref[0].astype(jnp.bfloat16),
                preferred_element_type=jnp.float32)
    y = jnp.where(f == 0, y + b2r_ref[0], y)

    @pl.when(f == 0)
    def _init():
        acc_ref[rows, :] = y

    @pl.when(f != 0)
    def _acc():
        acc_ref[rows, :] += y

    @pl.when(f == NF - 1)
    def _fin():
        ys_ref[rows, :] = (ws_ref[rows, :]
                           * acc_ref[rows, :]).astype(jnp.bfloat16)


def _grouped(te, scat_col, x2, ws2, W1_r, b1_r, W2_r, b2_r):
    grid_spec = pltpu.PrefetchScalarGridSpec(
        num_scalar_prefetch=1,
        grid=(NF, GMAX),
        in_specs=[pl.BlockSpec((PMAX, 1), lambda f, g, te: (0, 0)),
                  pl.BlockSpec((S, D), lambda f, g, te: (0, 0)),
                  pl.BlockSpec((PMAX, 1), lambda f, g, te: (0, 0)),
                  pl.BlockSpec((1, D, TF), lambda f, g, te: (te[g], 0, f)),
                  pl.BlockSpec((1, 1, TF), lambda f, g, te: (te[g], 0, f)),
                  pl.BlockSpec((1, TF, D), lambda f, g, te: (te[g], f, 0)),
                  pl.BlockSpec((1, 1, D), lambda f, g, te: (te[g], 0, 0))],
        out_specs=pl.BlockSpec((PMAX, D), lambda f, g, te: (0, 0)),
        scratch_shapes=[pltpu.VMEM((PMAX, D), jnp.float32),
                        pltpu.VMEM((S, D), jnp.bfloat16),
                        pltpu.VMEM((PMAX, D), jnp.bfloat16)],
    )
    return pl.pallas_call(
        _group_body,
        grid_spec=grid_spec,
        out_shape=jax.ShapeDtypeStruct((PMAX, D), jnp.bfloat16),
        compiler_params=pltpu.CompilerParams(
            dimension_semantics=("arbitrary", "arbitrary")),
    )(te, scat_col, x2, ws2, W1_r, b1_r.reshape(E, 1, F), W2_r,
      b2_r.reshape(E, 1, D))


def _unsort_body(scat_ref, ys_ref, base_ref, out_ref):
    t = pl.program_id(0)
    tio = lax.broadcasted_iota(jnp.int32, (TT, 1), 0) + t * TT
    acc = base_ref[...]
    for s in range(GMAX):
        oh = (scat_ref[:, pl.ds(s * TT, TT)] == tio).astype(jnp.bfloat16)
        acc += jnp.dot(oh, ys_ref[pl.ds(s * TT, TT), :],
                       preferred_element_type=jnp.float32)
    out_ref[...] = acc


def _combine(scat_row, ys, base):
    return pl.pallas_call(
        _unsort_body,
        grid=(NT,),
        in_specs=[pl.BlockSpec((1, PMAX), lambda t: (0, 0)),
                  pl.BlockSpec((PMAX, D), lambda t: (0, 0)),
                  pl.BlockSpec((TT, D), lambda t: (t, 0))],
        out_specs=pl.BlockSpec((TT, D), lambda t: (t, 0)),
        out_shape=jax.ShapeDtypeStruct((S, D), jnp.float32),
        compiler_params=pltpu.CompilerParams(
            dimension_semantics=("arbitrary",)),
    )(scat_row, ys, base)


def kernel(x, Wg, bg, W1_sh, b1_sh, W2_sh, b2_sh, W1_r, b1_r, W2_r, b2_r):
    B = x.shape[0]
    x2 = x.reshape(S, D)
    wg_p = jnp.zeros((D, 128), jnp.float32).at[:, :E].set(Wg)
    bg_p = jnp.zeros((1, 128), jnp.float32).at[0, :E].set(bg)

    eid2, wgt2 = _gate(x2, wg_p, bg_p)
    scat, ws, te = _route(eid2.reshape(S), wgt2.reshape(S))
    base = _shared(x2, W1_sh, b1_sh.reshape(1, F), W2_sh, b2_sh.reshape(1, D))
    ys = _grouped(te, scat.reshape(PMAX, 1), x2, ws.reshape(PMAX, 1),
                  W1_r, b1_r, W2_r, b2_r)
    out = _combine(scat.reshape(1, PMAX), ys, base)
    return out.reshape(B, S, D)
